# baseline (speedup 1.0000x reference)
import numpy as np
import jax
import jax.numpy as jnp
from jax import lax
from jax.experimental import pallas as pl
from jax.experimental.pallas import tpu as pltpu
from jax.experimental.pallas import tpu_sc as plsc

_EXTRA_IDXS = np.array([
    9120, 9929, 9448, 616, 6,
    5770, 5780, 8846, 8463, 8474, 8635,
    5361, 4933, 5058, 5169, 5286,
    8079, 7669, 7794, 7905, 8022
], dtype=np.int32)

_B, _V, _C = 1024, 10475, 3
_J, _E = 55, 21
_OW = _J + _E
_NBB = _B // 128
_NACT = _C * _NBB


def _sc_body(vt, jt, out, vbuf, jbuf, blk):
    wid = lax.axis_index("s") * 2 + lax.axis_index("c")
    c = wid // _NBB
    bb = wid % _NBB

    @pl.when(wid < _NACT)
    def _():
        pltpu.sync_copy(jt.at[c, :, pl.ds(bb * 128, 128)], jbuf)
        for j in range(_E):
            band = int(_EXTRA_IDXS[j]) // 8
            pltpu.sync_copy(
                vt.at[c, pl.ds(band * 8, 8), pl.ds(bb * 128, 128)],
                vbuf.at[pl.ds(j * 8, 8)])
        for r in range(_J):
            for l in range(8):
                blk[r, pl.ds(l * 16, 16)] = jbuf[r, pl.ds(l * 16, 16)]
        for j in range(_E):
            s = j * 8 + int(_EXTRA_IDXS[j]) % 8
            for l in range(8):
                blk[_J + j, pl.ds(l * 16, 16)] = vbuf[s, pl.ds(l * 16, 16)]
        pltpu.sync_copy(blk, out.at[c, :, pl.ds(bb * 128, 128)])


_mesh = plsc.VectorSubcoreMesh(core_axis_name="c", subcore_axis_name="s")
_run = pl.kernel(
    _sc_body,
    out_type=jax.ShapeDtypeStruct((_C, _OW, _B), jnp.float32),
    mesh=_mesh,
    scratch_types=[
        pltpu.VMEM((_E * 8, 128), jnp.float32),
        pltpu.VMEM((_J, 128), jnp.float32),
        pltpu.VMEM((_OW, 128), jnp.float32),
    ],
    compiler_params=pltpu.CompilerParams(
        use_tc_tiling_on_sc=True, needs_layout_passes=False),
)


@jax.jit
def kernel(vertices, joints):
    vt = jnp.transpose(vertices, (2, 1, 0))
    jt = jnp.transpose(joints, (2, 1, 0))
    return jnp.transpose(_run(vt, jt), (2, 1, 0))


# speedup vs baseline: 1.4935x; 1.4935x over previous
import numpy as np
import jax
import jax.numpy as jnp
from jax import lax
from jax.experimental import pallas as pl
from jax.experimental.pallas import tpu as pltpu
from jax.experimental.pallas import tpu_sc as plsc

_EXTRA_IDXS = np.array([
    9120, 9929, 9448, 616, 6,
    5770, 5780, 8846, 8463, 8474, 8635,
    5361, 4933, 5058, 5169, 5286,
    8079, 7669, 7794, 7905, 8022
], dtype=np.int32)

_B, _V, _C = 1024, 10475, 3
_J, _E = 55, 21
_OW = _J + _E
_NBB = _B // 128
_NACT = _C * _NBB


def _sc_body(vt, jt, out, vbuf, jbuf, blk, jsem, vsem):
    wid = lax.axis_index("s") * 2 + lax.axis_index("c")
    c = wid // _NBB
    bb = wid % _NBB

    @pl.when(wid < _NACT)
    def _():
        jcp = pltpu.async_copy(jt.at[c, :, pl.ds(bb * 128, 128)], jbuf, jsem)
        vcps = []
        for j in range(_E):
            band = int(_EXTRA_IDXS[j]) // 8
            vcps.append(pltpu.async_copy(
                vt.at[c, pl.ds(band * 8, 8), pl.ds(bb * 128, 128)],
                vbuf.at[pl.ds(j * 8, 8)], vsem))
        jcp.wait()
        for r in range(_J):
            for l in range(8):
                blk[r, pl.ds(l * 16, 16)] = jbuf[r, pl.ds(l * 16, 16)]
        for cp in vcps:
            cp.wait()
        for j in range(_E):
            s = j * 8 + int(_EXTRA_IDXS[j]) % 8
            for l in range(8):
                blk[_J + j, pl.ds(l * 16, 16)] = vbuf[s, pl.ds(l * 16, 16)]
        pltpu.sync_copy(blk, out.at[c, :, pl.ds(bb * 128, 128)])


_mesh = plsc.VectorSubcoreMesh(core_axis_name="c", subcore_axis_name="s")
_run = pl.kernel(
    _sc_body,
    out_type=jax.ShapeDtypeStruct((_C, _OW, _B), jnp.float32),
    mesh=_mesh,
    scratch_types=[
        pltpu.VMEM((_E * 8, 128), jnp.float32),
        pltpu.VMEM((_J, 128), jnp.float32),
        pltpu.VMEM((_OW, 128), jnp.float32),
        pltpu.SemaphoreType.DMA,
        pltpu.SemaphoreType.DMA,
    ],
    compiler_params=pltpu.CompilerParams(
        use_tc_tiling_on_sc=True, needs_layout_passes=False),
)


@jax.jit
def kernel(vertices, joints):
    vt = jnp.transpose(vertices, (2, 1, 0))
    jt = jnp.transpose(joints, (2, 1, 0))
    return jnp.transpose(_run(vt, jt), (2, 1, 0))


# speedup vs baseline: 12.1090x; 8.1078x over previous
import numpy as np
import jax
import jax.numpy as jnp
from jax.experimental import pallas as pl

_EXTRA_IDXS = np.array([
    9120, 9929, 9448, 616, 6,
    5770, 5780, 8846, 8463, 8474, 8635,
    5361, 4933, 5058, 5169, 5286,
    8079, 7669, 7794, 7905, 8022
], dtype=np.int32)

_B, _V, _C = 1024, 10475, 3
_J, _E = 55, 21


def _body(*refs):
    jt_ref = refs[0]
    vrefs = refs[1:1 + _E]
    out_ref = refs[1 + _E]
    out_ref[:, 0:_J, :] = jt_ref[:]
    for j in range(_E):
        s = int(_EXTRA_IDXS[j]) % 8
        out_ref[:, _J + j:_J + j + 1, :] = vrefs[j][:, s:s + 1, :]


@jax.jit
def kernel(vertices, joints):
    vt = jnp.transpose(vertices, (2, 1, 0))
    jt = jnp.transpose(joints, (2, 1, 0))
    in_specs = [pl.BlockSpec((_C, _J, _B), lambda i: (0, 0, 0))]
    for j in range(_E):
        blk = int(_EXTRA_IDXS[j]) // 8
        in_specs.append(
            pl.BlockSpec((_C, 8, _B), lambda i, _blk=blk: (0, _blk, 0)))
    out_t = pl.pallas_call(
        _body,
        grid=(1,),
        in_specs=in_specs,
        out_specs=pl.BlockSpec((_C, _J + _E, _B), lambda i: (0, 0, 0)),
        out_shape=jax.ShapeDtypeStruct((_C, _J + _E, _B), jnp.float32),
    )(jt, *([vt] * _E))
    return jnp.transpose(out_t, (2, 1, 0))


# speedup vs baseline: 12.7447x; 1.0525x over previous
import numpy as np
import jax
import jax.numpy as jnp
from jax.experimental import pallas as pl

_EXTRA_IDXS = np.array([
    9120, 9929, 9448, 616, 6,
    5770, 5780, 8846, 8463, 8474, 8635,
    5361, 4933, 5058, 5169, 5286,
    8079, 7669, 7794, 7905, 8022
], dtype=np.int32)

_B, _V, _C = 1024, 10475, 3
_J, _E = 55, 21


def _body(*refs):
    jt_ref = refs[0]
    vrefs = refs[1:1 + _E]
    out_ref = refs[1 + _E]
    out_ref[:, 0:_J, :] = jt_ref[:]
    for j in range(_E):
        s = int(_EXTRA_IDXS[j]) % 8
        out_ref[:, _J + j:_J + j + 1, :] = vrefs[j][:, s:s + 1, :]


@jax.jit
def kernel(vertices, joints):
    vt = jnp.transpose(vertices, (2, 1, 0))
    jt = jnp.transpose(joints, (2, 1, 0))
    lb = _B // 2
    in_specs = [pl.BlockSpec((_C, _J, lb), lambda i: (0, 0, i))]
    for j in range(_E):
        blk = int(_EXTRA_IDXS[j]) // 8
        in_specs.append(
            pl.BlockSpec((_C, 8, lb), lambda i, _blk=blk: (0, _blk, i)))
    out_t = pl.pallas_call(
        _body,
        grid=(2,),
        in_specs=in_specs,
        out_specs=pl.BlockSpec((_C, _J + _E, lb), lambda i: (0, 0, i)),
        out_shape=jax.ShapeDtypeStruct((_C, _J + _E, _B), jnp.float32),
    )(jt, *([vt] * _E))
    return jnp.transpose(out_t, (2, 1, 0))
